# baseline (speedup 1.0000x reference)
import functools

import jax
import jax.numpy as jnp
from jax import lax
from jax.experimental import pallas as pl
from jax.experimental.pallas import tpu as pltpu
from jax.experimental.pallas import tpu_sc as plsc

N = 262144
V = 16 * 16 * 16 * 16
DPAD = 9
LANES = 16

NUM_CORES = 2
NUM_SUBCORES = 16
NW = NUM_CORES * NUM_SUBCORES
B_W = N // NW
CHUNK = 2048
NCHUNK = B_W // CHUNK


def _gather_body(ax_hbm, ay_hbm, th_hbm, ph_hbm, lut_hbm, out_hbm,
                 ax_v, ay_v, th_v, ph_v, lin_v, rows_v, sem):
    wid = lax.axis_index("s") * NUM_CORES + lax.axis_index("c")
    base = wid * B_W

    for c in range(NCHUNK):
        off = base + c * CHUNK
        pltpu.sync_copy(ax_hbm.at[pl.ds(off, CHUNK)], ax_v)
        pltpu.sync_copy(ay_hbm.at[pl.ds(off, CHUNK)], ay_v)
        pltpu.sync_copy(th_hbm.at[pl.ds(off, CHUNK)], th_v)
        pltpu.sync_copy(ph_hbm.at[pl.ds(off, CHUNK)], ph_v)

        def body(i, _):
            s = pl.ds(i * LANES, LANES)
            lin = (
                (ax_v[s] << 12) | (ay_v[s] << 8) | (th_v[s] << 4) | ph_v[s]
            )
            lin_v[s] = lin
            return _

        lax.fori_loop(0, CHUNK // LANES, body, None)

        pltpu.async_copy(lut_hbm.at[lin_v], rows_v, sem).wait()
        pltpu.sync_copy(rows_v, out_hbm.at[pl.ds(off, CHUNK)])


@functools.partial(jax.jit, static_argnums=())
def kernel(alphax_idx, alphay_idx, theta_idx, phi_idx, LUT):
    ax = alphax_idx.astype(jnp.int32)
    ay = alphay_idx.astype(jnp.int32)
    th = theta_idx.astype(jnp.int32)
    ph = phi_idx.astype(jnp.int32)

    lut_rows = LUT.reshape(V, 9)

    mesh = plsc.VectorSubcoreMesh(core_axis_name="c", subcore_axis_name="s")
    out = pl.kernel(
        _gather_body,
        mesh=mesh,
        compiler_params=pltpu.CompilerParams(use_tc_tiling_on_sc=False),
        out_type=jax.ShapeDtypeStruct((N, DPAD), jnp.float32),
        scratch_types=[
            pltpu.VMEM((CHUNK,), jnp.int32),
            pltpu.VMEM((CHUNK,), jnp.int32),
            pltpu.VMEM((CHUNK,), jnp.int32),
            pltpu.VMEM((CHUNK,), jnp.int32),
            pltpu.VMEM((CHUNK,), jnp.int32),
            pltpu.VMEM((CHUNK, DPAD), jnp.float32),
            pltpu.SemaphoreType.DMA,
        ],
    )(ax, ay, th, ph, lut_rows)

    return out.reshape(N, 3, 3)


# speedup vs baseline: 1.4845x; 1.4845x over previous
import functools

import jax
import jax.numpy as jnp
from jax import lax
from jax.experimental import pallas as pl
from jax.experimental.pallas import tpu as pltpu
from jax.experimental.pallas import tpu_sc as plsc

N = 262144
V = 16 * 16 * 16 * 16
D = 9
LANES = 16

NUM_CORES = 2
NUM_SUBCORES = 16
NW = NUM_CORES * NUM_SUBCORES
B_W = N // NW
CHUNK = 2048
NCHUNK = B_W // CHUNK
NBLK = CHUNK // 128


def _gather_body(ax_hbm, ay_hbm, th_hbm, ph_hbm, lut_hbm, out_hbm,
                 ax_v, ay_v, th_v, ph_v, lin_v, rows_v, planes_v, sem):
    wid = lax.axis_index("s") * NUM_CORES + lax.axis_index("c")
    base = wid * B_W
    iota = lax.iota(jnp.int32, LANES)

    for c in range(NCHUNK):
        off = base + c * CHUNK
        pltpu.sync_copy(ax_hbm.at[pl.ds(off, CHUNK)], ax_v)
        pltpu.sync_copy(ay_hbm.at[pl.ds(off, CHUNK)], ay_v)
        pltpu.sync_copy(th_hbm.at[pl.ds(off, CHUNK)], th_v)
        pltpu.sync_copy(ph_hbm.at[pl.ds(off, CHUNK)], ph_v)

        def lin_body(i, _):
            s = pl.ds(i * LANES, LANES)
            lin_v[s] = (ax_v[s] << 12) | (ay_v[s] << 8) | (th_v[s] << 4) | ph_v[s]
            return _

        lax.fori_loop(0, CHUNK // LANES, lin_body, None)

        pltpu.async_copy(lut_hbm.at[lin_v], rows_v, sem).wait()

        def tr_body(i, _):
            row_idx = iota + i * LANES
            blk = i // 8
            lane = (i % 8) * LANES
            for k in range(D):
                vals = plsc.load_gather(rows_v, [row_idx, iota * 0 + k])
                planes_v[k, blk, pl.ds(lane, LANES)] = vals
            return _

        lax.fori_loop(0, CHUNK // LANES, tr_body, None)

        t0 = off // 128
        for k in range(D):
            pltpu.sync_copy(planes_v.at[k],
                            out_hbm.at[k // 3, pl.ds(t0, NBLK), k % 3])


@functools.partial(jax.jit, static_argnums=())
def kernel(alphax_idx, alphay_idx, theta_idx, phi_idx, LUT):
    ax = alphax_idx.astype(jnp.int32)
    ay = alphay_idx.astype(jnp.int32)
    th = theta_idx.astype(jnp.int32)
    ph = phi_idx.astype(jnp.int32)

    lut_rows = LUT.reshape(V, D)

    mesh = plsc.VectorSubcoreMesh(core_axis_name="c", subcore_axis_name="s")
    out4 = pl.kernel(
        _gather_body,
        mesh=mesh,
        compiler_params=pltpu.CompilerParams(use_tc_tiling_on_sc=False, needs_layout_passes=False),
        out_type=jax.ShapeDtypeStruct((3, N // 128, 4, 128), jnp.float32),
        scratch_types=[
            pltpu.VMEM((CHUNK,), jnp.int32),
            pltpu.VMEM((CHUNK,), jnp.int32),
            pltpu.VMEM((CHUNK,), jnp.int32),
            pltpu.VMEM((CHUNK,), jnp.int32),
            pltpu.VMEM((CHUNK,), jnp.int32),
            pltpu.VMEM((CHUNK, D), jnp.float32),
            pltpu.VMEM((D, NBLK, 128), jnp.float32),
            pltpu.SemaphoreType.DMA,
        ],
    )(ax, ay, th, ph, lut_rows)

    return out4[:, :, :3, :].transpose(1, 3, 0, 2).reshape(N, 3, 3)


# speedup vs baseline: 1.4895x; 1.0033x over previous
import functools

import jax
import jax.numpy as jnp
from jax import lax
from jax.experimental import pallas as pl
from jax.experimental.pallas import tpu as pltpu
from jax.experimental.pallas import tpu_sc as plsc

N = 262144
V = 16 * 16 * 16 * 16
D = 9
LANES = 16

NUM_CORES = 2
NUM_SUBCORES = 16
NW = NUM_CORES * NUM_SUBCORES
B_W = N // NW
CHUNK = 2048
NCHUNK = B_W // CHUNK
NBLK = CHUNK // 128


def _gather_body(ax_hbm, ay_hbm, th_hbm, ph_hbm, lut_hbm, out_hbm,
                 ax_v, ay_v, th_v, ph_v, lin_v, rows_v, planes_v, sem):
    wid = lax.axis_index("s") * NUM_CORES + lax.axis_index("c")
    base = wid * B_W
    iota = lax.iota(jnp.int32, LANES)

    for c in range(NCHUNK):
        off = base + c * CHUNK
        pltpu.sync_copy(ax_hbm.at[pl.ds(off, CHUNK)], ax_v)
        pltpu.sync_copy(ay_hbm.at[pl.ds(off, CHUNK)], ay_v)
        pltpu.sync_copy(th_hbm.at[pl.ds(off, CHUNK)], th_v)
        pltpu.sync_copy(ph_hbm.at[pl.ds(off, CHUNK)], ph_v)

        def lin_body(i, _):
            s = pl.ds(i * LANES, LANES)
            lin_v[s] = (ax_v[s] << 12) | (ay_v[s] << 8) | (th_v[s] << 4) | ph_v[s]
            return _

        lax.fori_loop(0, CHUNK // LANES, lin_body, None)

        pltpu.async_copy(lut_hbm.at[lin_v], rows_v, sem).wait()

        def tr_body(i, _):
            row_idx = iota + i * LANES
            blk = i // 8
            lane = (i % 8) * LANES
            for k in range(D):
                vals = plsc.load_gather(rows_v, [row_idx, iota * 0 + k])
                planes_v[k, blk, pl.ds(lane, LANES)] = vals
            return _

        lax.fori_loop(0, CHUNK // LANES, tr_body, None)

        t0 = off // 128
        for k in range(D):
            pltpu.sync_copy(planes_v.at[k],
                            out_hbm.at[k // 3, pl.ds(t0, NBLK), k % 3])


@functools.partial(jax.jit, static_argnums=())
def kernel(alphax_idx, alphay_idx, theta_idx, phi_idx, LUT):
    ax = alphax_idx.astype(jnp.int32)
    ay = alphay_idx.astype(jnp.int32)
    th = theta_idx.astype(jnp.int32)
    ph = phi_idx.astype(jnp.int32)

    lut_rows = jnp.pad(LUT.reshape(V, D), ((0, 0), (0, LANES - D)))

    mesh = plsc.VectorSubcoreMesh(core_axis_name="c", subcore_axis_name="s")
    out4 = pl.kernel(
        _gather_body,
        mesh=mesh,
        compiler_params=pltpu.CompilerParams(use_tc_tiling_on_sc=False, needs_layout_passes=False),
        out_type=jax.ShapeDtypeStruct((3, N // 128, 4, 128), jnp.float32),
        scratch_types=[
            pltpu.VMEM((CHUNK,), jnp.int32),
            pltpu.VMEM((CHUNK,), jnp.int32),
            pltpu.VMEM((CHUNK,), jnp.int32),
            pltpu.VMEM((CHUNK,), jnp.int32),
            pltpu.VMEM((CHUNK,), jnp.int32),
            pltpu.VMEM((CHUNK, LANES), jnp.float32),
            pltpu.VMEM((D, NBLK, 128), jnp.float32),
            pltpu.SemaphoreType.DMA,
        ],
    )(ax, ay, th, ph, lut_rows)

    return out4[:, :, :3, :].transpose(1, 3, 0, 2).reshape(N, 3, 3)


# speedup vs baseline: 1.6449x; 1.1043x over previous
import functools

import jax
import jax.numpy as jnp
from jax import lax
from jax.experimental import pallas as pl
from jax.experimental.pallas import tpu as pltpu
from jax.experimental.pallas import tpu_sc as plsc

N = 262144
V = 16 * 16 * 16 * 16
D = 9
LANES = 16

NUM_CORES = 2
NUM_SUBCORES = 16
NW = NUM_CORES * NUM_SUBCORES
B_W = N // NW
CHUNK = 1024
NCHUNK = B_W // CHUNK
NBLK = CHUNK // 128


def _gather_body(ax_hbm, ay_hbm, th_hbm, ph_hbm, lut_hbm, out_hbm,
                 ax_v, ay_v, th_v, ph_v, lin_v, rows2, planes2,
                 gsem0, gsem1, wsem0, wsem1):
    wid = lax.axis_index("s") * NUM_CORES + lax.axis_index("c")
    base = wid * B_W
    gsems = (gsem0, gsem1)
    wsems = (wsem0, wsem1)
    iota = lax.iota(jnp.int32, LANES)

    pltpu.sync_copy(ax_hbm.at[pl.ds(base, B_W)], ax_v)
    pltpu.sync_copy(ay_hbm.at[pl.ds(base, B_W)], ay_v)
    pltpu.sync_copy(th_hbm.at[pl.ds(base, B_W)], th_v)
    pltpu.sync_copy(ph_hbm.at[pl.ds(base, B_W)], ph_v)

    def lin_body(i, _):
        s = pl.ds(i * LANES, LANES)
        cc = i // (CHUNK // LANES)
        ss = pl.ds((i % (CHUNK // LANES)) * LANES, LANES)
        lin_v[cc, ss] = (ax_v[s] << 12) | (ay_v[s] << 8) | (th_v[s] << 4) | ph_v[s]
        return _

    lax.fori_loop(0, B_W // LANES, lin_body, None)

    def fire_gather(c):
        s = c % 2
        return pltpu.async_copy(lut_hbm.at[lin_v.at[c]], rows2.at[s], gsems[s])

    def transpose_chunk(s):
        def tr_body(i, _):
            row_idx = iota + i * LANES
            blk = i // 8
            lane = (i % 8) * LANES
            for k in range(D):
                vals = plsc.load_gather(rows2.at[s], [row_idx, iota * 0 + k])
                planes2[s, k, blk, pl.ds(lane, LANES)] = vals
            return _

        lax.fori_loop(0, CHUNK // LANES, tr_body, None)

    def fire_writes(c):
        s = c % 2
        t0 = (base + c * CHUNK) // 128
        return [
            pltpu.async_copy(planes2.at[s, k],
                             out_hbm.at[k // 3, pl.ds(t0, NBLK), k % 3],
                             wsems[s])
            for k in range(D)
        ]

    wdesc = [None, None]
    g = fire_gather(0)
    for c in range(NCHUNK):
        g.wait()
        if c + 1 < NCHUNK:
            g = fire_gather(c + 1)
        s = c % 2
        if wdesc[s] is not None:
            for w in wdesc[s]:
                w.wait()
            wdesc[s] = None
        transpose_chunk(s)
        wdesc[s] = fire_writes(c)
    for sl in (0, 1):
        if wdesc[sl] is not None:
            for w in wdesc[sl]:
                w.wait()


@functools.partial(jax.jit, static_argnums=())
def kernel(alphax_idx, alphay_idx, theta_idx, phi_idx, LUT):
    ax = alphax_idx.astype(jnp.int32)
    ay = alphay_idx.astype(jnp.int32)
    th = theta_idx.astype(jnp.int32)
    ph = phi_idx.astype(jnp.int32)

    lut_rows = jnp.pad(LUT.reshape(V, D), ((0, 0), (0, LANES - D)))

    mesh = plsc.VectorSubcoreMesh(core_axis_name="c", subcore_axis_name="s")
    out4 = pl.kernel(
        _gather_body,
        mesh=mesh,
        compiler_params=pltpu.CompilerParams(
            use_tc_tiling_on_sc=False, needs_layout_passes=False),
        out_type=jax.ShapeDtypeStruct((3, N // 128, 4, 128), jnp.float32),
        scratch_types=[
            pltpu.VMEM((B_W,), jnp.int32),
            pltpu.VMEM((B_W,), jnp.int32),
            pltpu.VMEM((B_W,), jnp.int32),
            pltpu.VMEM((B_W,), jnp.int32),
            pltpu.VMEM((NCHUNK, CHUNK), jnp.int32),
            pltpu.VMEM((2, CHUNK, LANES), jnp.float32),
            pltpu.VMEM((2, D, NBLK, 128), jnp.float32),
            pltpu.SemaphoreType.DMA,
            pltpu.SemaphoreType.DMA,
            pltpu.SemaphoreType.DMA,
            pltpu.SemaphoreType.DMA,
        ],
    )(ax, ay, th, ph, lut_rows)

    return out4[:, :, :3, :].transpose(1, 3, 0, 2).reshape(N, 3, 3)


# speedup vs baseline: 2.7460x; 1.6694x over previous
import functools

import jax
import jax.numpy as jnp
from jax import lax
from jax.experimental import pallas as pl
from jax.experimental.pallas import tpu as pltpu
from jax.experimental.pallas import tpu_sc as plsc

N = 262144
V = 16 * 16 * 16 * 16
D = 9
LANES = 16

NUM_CORES = 2
NUM_SUBCORES = 16
NW = NUM_CORES * NUM_SUBCORES
B_W = N // NW
CHUNK = 1024
NCHUNK = B_W // CHUNK
NBLK = CHUNK // 128

NPAIR_W = 256 // NUM_SUBCORES


def _gather_body(ax_hbm, ay_hbm, th_hbm, ph_hbm, lutc_hbm,
                 out_hbm, table_hbm,
                 ax_v, ay_v, th_v, ph_v, lin_v, rows2, planes2,
                 vbuf, rows256,
                 gsem0, gsem1, wsem0, wsem1, tsem):
    sc = lax.axis_index("c")
    sub = lax.axis_index("s")
    wid = sub * NUM_CORES + sc
    base = wid * B_W
    gsems = (gsem0, gsem1)
    wsems = (wsem0, wsem1)
    iota = lax.iota(jnp.int32, LANES)

    def pair_body(j, _):
        pair = sub * NPAIR_W + j
        pltpu.sync_copy(lutc_hbm.at[pl.ds(pair * D, D)], vbuf)

        def tp_body(p, _):
            rowi = iota + p * LANES
            for k in range(D):
                plsc.store_scatter(rows256, [rowi, iota * 0 + k],
                                   vbuf[k, pl.ds(p * LANES, LANES)])
            return _

        lax.fori_loop(0, 256 // LANES, tp_body, None)
        pltpu.sync_copy(rows256, table_hbm.at[sc, pl.ds(pair * 256, 256)])
        return _

    lax.fori_loop(0, NPAIR_W, pair_body, None)

    pltpu.sync_copy(ax_hbm.at[pl.ds(base, B_W)], ax_v)
    pltpu.sync_copy(ay_hbm.at[pl.ds(base, B_W)], ay_v)
    pltpu.sync_copy(th_hbm.at[pl.ds(base, B_W)], th_v)
    pltpu.sync_copy(ph_hbm.at[pl.ds(base, B_W)], ph_v)

    def lin_body(i, _):
        s = pl.ds(i * LANES, LANES)
        cc = i // (CHUNK // LANES)
        ss = pl.ds((i % (CHUNK // LANES)) * LANES, LANES)
        lin_v[cc, ss] = (ax_v[s] << 12) | (ay_v[s] << 8) | (th_v[s] << 4) | ph_v[s]
        return _

    lax.fori_loop(0, B_W // LANES, lin_body, None)

    plsc.subcore_barrier()

    my_table = table_hbm.at[sc]

    def fire_gather(c):
        s = c % 2
        return pltpu.async_copy(my_table.at[lin_v.at[c]], rows2.at[s], gsems[s])

    def transpose_chunk(s):
        def tr_body(i, _):
            row_idx = iota + i * LANES
            blk = i // 8
            lane = (i % 8) * LANES
            for k in range(D):
                vals = plsc.load_gather(rows2.at[s], [row_idx, iota * 0 + k])
                planes2[s, k, blk, pl.ds(lane, LANES)] = vals
            return _

        lax.fori_loop(0, CHUNK // LANES, tr_body, None)

    def fire_writes(c):
        s = c % 2
        t0 = (base + c * CHUNK) // 128
        return [
            pltpu.async_copy(planes2.at[s, k],
                             out_hbm.at[k // 3, pl.ds(t0, NBLK), k % 3],
                             wsems[s])
            for k in range(D)
        ]

    wdesc = [None, None]
    g = fire_gather(0)
    for c in range(NCHUNK):
        g.wait()
        if c + 1 < NCHUNK:
            g = fire_gather(c + 1)
        s = c % 2
        if wdesc[s] is not None:
            for w in wdesc[s]:
                w.wait()
            wdesc[s] = None
        transpose_chunk(s)
        wdesc[s] = fire_writes(c)
    for sl in (0, 1):
        if wdesc[sl] is not None:
            for w in wdesc[sl]:
                w.wait()


@functools.partial(jax.jit, static_argnums=())
def kernel(alphax_idx, alphay_idx, theta_idx, phi_idx, LUT):
    ax = alphax_idx.astype(jnp.int32)
    ay = alphay_idx.astype(jnp.int32)
    th = theta_idx.astype(jnp.int32)
    ph = phi_idx.astype(jnp.int32)

    lutc = jnp.transpose(LUT, (0, 1, 4, 5, 2, 3)).reshape(2304, 256)

    mesh = plsc.VectorSubcoreMesh(core_axis_name="c", subcore_axis_name="s")
    out4, _ = pl.kernel(
        _gather_body,
        mesh=mesh,
        compiler_params=pltpu.CompilerParams(
            use_tc_tiling_on_sc=False, needs_layout_passes=False),
        out_type=(
            jax.ShapeDtypeStruct((3, N // 128, 4, 128), jnp.float32),
            jax.ShapeDtypeStruct((NUM_CORES, V, LANES), jnp.float32),
        ),
        scratch_types=[
            pltpu.VMEM((B_W,), jnp.int32),
            pltpu.VMEM((B_W,), jnp.int32),
            pltpu.VMEM((B_W,), jnp.int32),
            pltpu.VMEM((B_W,), jnp.int32),
            pltpu.VMEM((NCHUNK, CHUNK), jnp.int32),
            pltpu.VMEM((2, CHUNK, LANES), jnp.float32),
            pltpu.VMEM((2, D, NBLK, 128), jnp.float32),
            pltpu.VMEM((D, 256), jnp.float32),
            pltpu.VMEM((256, LANES), jnp.float32),
            pltpu.SemaphoreType.DMA,
            pltpu.SemaphoreType.DMA,
            pltpu.SemaphoreType.DMA,
            pltpu.SemaphoreType.DMA,
            pltpu.SemaphoreType.DMA,
        ],
    )(ax, ay, th, ph, lutc)

    return out4[:, :, :3, :].transpose(1, 3, 0, 2).reshape(N, 3, 3)


# speedup vs baseline: 2.9449x; 1.0724x over previous
import functools

import jax
import jax.numpy as jnp
from jax import lax
from jax.experimental import pallas as pl
from jax.experimental.pallas import tpu as pltpu
from jax.experimental.pallas import tpu_sc as plsc

N = 262144
V = 16 * 16 * 16 * 16
D = 9
LANES = 16

NUM_CORES = 2
NUM_SUBCORES = 16
NW = NUM_CORES * NUM_SUBCORES
B_W = N // NW
CHUNK = 1024
NCHUNK = B_W // CHUNK
NBLK = CHUNK // 128

NPAIR_W = 256 // NUM_SUBCORES


def _gather_body(ax_hbm, ay_hbm, th_hbm, ph_hbm, lutc_hbm,
                 out_hbm, table_hbm,
                 ax_v, ay_v, th_v, ph_v, lin_v, rows2, planes2,
                 vbuf, rows256,
                 gsem0, gsem1, wsem0, wsem1,
                 asem0, asem1, bsem0, bsem1):
    sc = lax.axis_index("c")
    sub = lax.axis_index("s")
    wid = sub * NUM_CORES + sc
    base = wid * B_W
    gsems = (gsem0, gsem1)
    wsems = (wsem0, wsem1)
    asems = (asem0, asem1)
    bsems = (bsem0, bsem1)
    iota = lax.iota(jnp.int32, LANES)

    def fire_pair_load(j):
        s = j % 2
        pair = sub * NPAIR_W + j
        return pltpu.async_copy(lutc_hbm.at[pl.ds(pair * D, D)],
                                vbuf.at[s], asems[s])

    def fire_pair_store(j):
        s = j % 2
        pair = sub * NPAIR_W + j
        return pltpu.async_copy(rows256.at[s],
                                table_hbm.at[sc, pl.ds(pair * 256, 256)],
                                bsems[s])

    sdesc = [None, None]
    ld = fire_pair_load(0)
    for j in range(NPAIR_W):
        s = j % 2
        ld.wait()
        if j + 1 < NPAIR_W:
            ld = fire_pair_load(j + 1)
        if sdesc[s] is not None:
            sdesc[s].wait()
            sdesc[s] = None

        def tp_body(p, _):
            rowi = iota + p * LANES
            for k in range(D):
                plsc.store_scatter(rows256.at[s], [rowi, iota * 0 + k],
                                   vbuf[s, k, pl.ds(p * LANES, LANES)])
            return _

        lax.fori_loop(0, 256 // LANES, tp_body, None)
        sdesc[s] = fire_pair_store(j)
    for sl in (0, 1):
        if sdesc[sl] is not None:
            sdesc[sl].wait()

    pltpu.sync_copy(ax_hbm.at[pl.ds(base, B_W)], ax_v)
    pltpu.sync_copy(ay_hbm.at[pl.ds(base, B_W)], ay_v)
    pltpu.sync_copy(th_hbm.at[pl.ds(base, B_W)], th_v)
    pltpu.sync_copy(ph_hbm.at[pl.ds(base, B_W)], ph_v)

    def lin_body(i, _):
        s = pl.ds(i * LANES, LANES)
        cc = i // (CHUNK // LANES)
        ss = pl.ds((i % (CHUNK // LANES)) * LANES, LANES)
        lin_v[cc, ss] = (ax_v[s] << 12) | (ay_v[s] << 8) | (th_v[s] << 4) | ph_v[s]
        return _

    lax.fori_loop(0, B_W // LANES, lin_body, None)

    plsc.subcore_barrier()

    my_table = table_hbm.at[sc]

    def fire_gather(c):
        s = c % 2
        return pltpu.async_copy(my_table.at[lin_v.at[c]], rows2.at[s], gsems[s])

    def transpose_chunk(s):
        def tr_body(i2, _):
            for u in range(2):
                i = i2 * 2 + u
                row_idx = iota + i * LANES
                blk = i // 8
                lane = (i % 8) * LANES
                for k in range(D):
                    vals = plsc.load_gather(rows2.at[s], [row_idx, iota * 0 + k])
                    planes2[s, k, blk, pl.ds(lane, LANES)] = vals
            return _

        lax.fori_loop(0, CHUNK // LANES // 2, tr_body, None)

    def fire_writes(c):
        s = c % 2
        t0 = (base + c * CHUNK) // 128
        return [
            pltpu.async_copy(planes2.at[s, k],
                             out_hbm.at[k // 3, pl.ds(t0, NBLK), k % 3],
                             wsems[s])
            for k in range(D)
        ]

    wdesc = [None, None]
    g = fire_gather(0)
    for c in range(NCHUNK):
        g.wait()
        if c + 1 < NCHUNK:
            g = fire_gather(c + 1)
        s = c % 2
        if wdesc[s] is not None:
            for w in wdesc[s]:
                w.wait()
            wdesc[s] = None
        transpose_chunk(s)
        wdesc[s] = fire_writes(c)
    for sl in (0, 1):
        if wdesc[sl] is not None:
            for w in wdesc[sl]:
                w.wait()


@functools.partial(jax.jit, static_argnums=())
def kernel(alphax_idx, alphay_idx, theta_idx, phi_idx, LUT):
    ax = alphax_idx.astype(jnp.int32)
    ay = alphay_idx.astype(jnp.int32)
    th = theta_idx.astype(jnp.int32)
    ph = phi_idx.astype(jnp.int32)

    lutc = jnp.transpose(LUT, (0, 1, 4, 5, 2, 3)).reshape(2304, 256)

    mesh = plsc.VectorSubcoreMesh(core_axis_name="c", subcore_axis_name="s")
    out4, _ = pl.kernel(
        _gather_body,
        mesh=mesh,
        compiler_params=pltpu.CompilerParams(
            use_tc_tiling_on_sc=False, needs_layout_passes=False),
        out_type=(
            jax.ShapeDtypeStruct((3, N // 128, 4, 128), jnp.float32),
            jax.ShapeDtypeStruct((NUM_CORES, V, LANES), jnp.float32),
        ),
        scratch_types=[
            pltpu.VMEM((B_W,), jnp.int32),
            pltpu.VMEM((B_W,), jnp.int32),
            pltpu.VMEM((B_W,), jnp.int32),
            pltpu.VMEM((B_W,), jnp.int32),
            pltpu.VMEM((NCHUNK, CHUNK), jnp.int32),
            pltpu.VMEM((2, CHUNK, LANES), jnp.float32),
            pltpu.VMEM((2, D, NBLK, 128), jnp.float32),
            pltpu.VMEM((2, D, 256), jnp.float32),
            pltpu.VMEM((2, 256, LANES), jnp.float32),
            pltpu.SemaphoreType.DMA,
            pltpu.SemaphoreType.DMA,
            pltpu.SemaphoreType.DMA,
            pltpu.SemaphoreType.DMA,
            pltpu.SemaphoreType.DMA,
            pltpu.SemaphoreType.DMA,
            pltpu.SemaphoreType.DMA,
            pltpu.SemaphoreType.DMA,
        ],
    )(ax, ay, th, ph, lutc)

    return out4[:, :, :3, :].transpose(1, 3, 0, 2).reshape(N, 3, 3)
